# baseline (speedup 1.0000x reference)
import jax
import jax.numpy as jnp
from jax.experimental import pallas as pl

MAX_NODES_ = 50
BB = 512


def _v_broadcast_kernel(fps_ref, wfp1_ref, bfp1_ref, wfp2_ref, bfp2_ref,
                        wv_ref, bv_ref, out_ref):
    e = jnp.dot(fps_ref[...], wfp1_ref[...],
                preferred_element_type=jnp.float32) + bfp1_ref[...]
    e = jnp.maximum(e, 0.0)
    e = jnp.dot(e, wfp2_ref[...],
                preferred_element_type=jnp.float32) + bfp2_ref[...]
    e = jnp.maximum(e, 0.0)
    v = jnp.dot(e, wv_ref[...],
                preferred_element_type=jnp.float32) + bv_ref[...]
    out_ref[...] = jnp.broadcast_to(v[:, None, :], out_ref.shape)


def kernel(x, edge_index, batch, fps, abeta_feature, W1, as1, ad1, b1,
           W2, as2, ad2, b2, W3, as3, ad3, b3, g1, be1, g2, be2, g3, be3,
           Wfp1, bfp1, Wfp2, bfp2, Wq, bq, Wk, bk, Wv, bv):
    bsz = batch.shape[0] // MAX_NODES_
    fps_dim = fps.shape[1]
    emb = Wfp1.shape[1]
    out_ch = Wv.shape[1]

    grid = (bsz // BB,)
    return pl.pallas_call(
        _v_broadcast_kernel,
        grid=grid,
        in_specs=[
            pl.BlockSpec((BB, fps_dim), lambda i: (i, 0)),
            pl.BlockSpec((fps_dim, emb), lambda i: (0, 0)),
            pl.BlockSpec((1, emb), lambda i: (0, 0)),
            pl.BlockSpec((emb, out_ch), lambda i: (0, 0)),
            pl.BlockSpec((1, out_ch), lambda i: (0, 0)),
            pl.BlockSpec((out_ch, out_ch), lambda i: (0, 0)),
            pl.BlockSpec((1, out_ch), lambda i: (0, 0)),
        ],
        out_specs=pl.BlockSpec((BB, MAX_NODES_, out_ch), lambda i: (i, 0, 0)),
        out_shape=jax.ShapeDtypeStruct((bsz, MAX_NODES_, out_ch), jnp.float32),
    )(fps, Wfp1, bfp1.reshape(1, emb), Wfp2, bfp2.reshape(1, out_ch),
      Wv, bv.reshape(1, out_ch))


# speedup vs baseline: 1.0601x; 1.0601x over previous
import jax
import jax.numpy as jnp
from jax.experimental import pallas as pl

MAX_NODES_ = 50
BB = 256


def _v_broadcast_kernel(fps_ref, wfp1_ref, bfp1_ref, wfp2_ref, bfp2_ref,
                        wv_ref, bv_ref, out_ref):
    e = jnp.dot(fps_ref[...], wfp1_ref[...],
                preferred_element_type=jnp.float32) + bfp1_ref[...]
    e = jnp.maximum(e, 0.0)
    e = jnp.dot(e, wfp2_ref[...],
                preferred_element_type=jnp.float32) + bfp2_ref[...]
    e = jnp.maximum(e, 0.0)
    v = jnp.dot(e, wv_ref[...],
                preferred_element_type=jnp.float32) + bv_ref[...]
    out_ref[...] = jnp.broadcast_to(v[:, None, :], out_ref.shape)


def kernel(x, edge_index, batch, fps, abeta_feature, W1, as1, ad1, b1,
           W2, as2, ad2, b2, W3, as3, ad3, b3, g1, be1, g2, be2, g3, be3,
           Wfp1, bfp1, Wfp2, bfp2, Wq, bq, Wk, bk, Wv, bv):
    bsz = batch.shape[0] // MAX_NODES_
    fps_dim = fps.shape[1]
    emb = Wfp1.shape[1]
    out_ch = Wv.shape[1]

    grid = (bsz // BB,)
    return pl.pallas_call(
        _v_broadcast_kernel,
        grid=grid,
        in_specs=[
            pl.BlockSpec((BB, fps_dim), lambda i: (i, 0)),
            pl.BlockSpec((fps_dim, emb), lambda i: (0, 0)),
            pl.BlockSpec((1, emb), lambda i: (0, 0)),
            pl.BlockSpec((emb, out_ch), lambda i: (0, 0)),
            pl.BlockSpec((1, out_ch), lambda i: (0, 0)),
            pl.BlockSpec((out_ch, out_ch), lambda i: (0, 0)),
            pl.BlockSpec((1, out_ch), lambda i: (0, 0)),
        ],
        out_specs=pl.BlockSpec((BB, MAX_NODES_, out_ch), lambda i: (i, 0, 0)),
        out_shape=jax.ShapeDtypeStruct((bsz, MAX_NODES_, out_ch), jnp.float32),
    )(fps, Wfp1, bfp1.reshape(1, emb), Wfp2, bfp2.reshape(1, out_ch),
      Wv, bv.reshape(1, out_ch))
